# initial kernel scaffold (unmeasured)
import jax
import jax.numpy as jnp
from jax import lax
from jax.experimental import pallas as pl
from jax.experimental.pallas import tpu as pltpu

N_DEV = 16
N_TOK = 2048
D_IN = 512
D_OUT = 1024
E_LOCAL = 8
N_EXP = 128
CHUNK = N_TOK // N_DEV
N_STEP = N_DEV - 1


def kernel(x, router_W, route_idx, expert_W):
    def body(x_ref, rw_ref, idx_ref, ew_ref, out_ref, comm_ref,
             rs_send, rs_recv, ag_send, ag_recv):
        my = lax.axis_index("i")
        right = lax.rem(my + 1, N_DEV)

        xv = x_ref[:, :]
        scores = jnp.dot(xv, rw_ref[:, :], preferred_element_type=jnp.float32)
        smax = jnp.max(scores, axis=1, keepdims=True)
        ex = jnp.exp(scores - smax)
        probs = ex / jnp.sum(ex, axis=1, keepdims=True)
        idx0 = idx_ref[:, 0:1]
        idx1 = idx_ref[:, 1:2]
        eids = lax.broadcasted_iota(jnp.int32, (N_TOK, N_EXP), 1)
        p0 = jnp.sum(jnp.where(eids == idx0, probs, 0.0), axis=1, keepdims=True)
        p1 = jnp.sum(jnp.where(eids == idx1, probs, 0.0), axis=1, keepdims=True)
        denom = p0 + p1
        g0 = p0 / denom
        g1 = p1 / denom

        acc = jnp.zeros((N_TOK, D_OUT), jnp.float32)
        for el in range(E_LOCAL):
            gid = my * E_LOCAL + el
            ge = jnp.where(idx0 == gid, g0, 0.0) + jnp.where(idx1 == gid, g1, 0.0)
            acc = acc + jnp.dot(xv * ge, ew_ref[el],
                                preferred_element_type=jnp.float32)
        out_ref[:, :] = acc

        def cs(c):
            return pl.ds(c * CHUNK, CHUNK)

        for s in range(N_STEP):
            send_c = lax.rem(my - s + N_DEV, N_DEV)
            recv_c = lax.rem(my - s - 1 + N_DEV, N_DEV)
            if s == 0:
                src = out_ref.at[cs(send_c), :]
            else:
                src = comm_ref.at[s - 1]
            rdma = pltpu.make_async_remote_copy(
                src_ref=src,
                dst_ref=comm_ref.at[s],
                send_sem=rs_send.at[s],
                recv_sem=rs_recv.at[s],
                device_id=(right,),
                device_id_type=pl.DeviceIdType.MESH,
            )
            rdma.start()
            rdma.wait()
            comm_ref[s] = comm_ref[s] + out_ref[cs(recv_c), :]

        red_c = lax.rem(my + 1, N_DEV)
        out_ref[cs(red_c), :] = comm_ref[N_STEP - 1]

        for t in range(N_STEP):
            send_c = lax.rem(my + 1 - t + N_DEV, N_DEV)
            rdma = pltpu.make_async_remote_copy(
                src_ref=out_ref.at[cs(send_c), :],
                dst_ref=out_ref.at[cs(send_c), :],
                send_sem=ag_send.at[t],
                recv_sem=ag_recv.at[t],
                device_id=(right,),
                device_id_type=pl.DeviceIdType.MESH,
            )
            rdma.start()
            rdma.wait()

    return pl.pallas_call(
        body,
        out_shape=jax.ShapeDtypeStruct((N_TOK, D_OUT), jnp.float32),
        in_specs=[pl.BlockSpec(memory_space=pltpu.VMEM)] * 4,
        out_specs=pl.BlockSpec(memory_space=pltpu.VMEM),
        scratch_shapes=[
            pltpu.VMEM((N_STEP, CHUNK, D_OUT), jnp.float32),
            pltpu.SemaphoreType.DMA((N_STEP,)),
            pltpu.SemaphoreType.DMA((N_STEP,)),
            pltpu.SemaphoreType.DMA((N_STEP,)),
            pltpu.SemaphoreType.DMA((N_STEP,)),
        ],
    )(x, router_W, route_idx, expert_W)


# baseline (device time: 280792 ns/iter reference)
import jax
import jax.numpy as jnp
from jax import lax
from jax.experimental import pallas as pl
from jax.experimental.pallas import tpu as pltpu

N_DEV = 16
N_TOK = 2048
D_IN = 512
D_OUT = 1024
E_LOCAL = 8
N_EXP = 128
CHUNK = N_TOK // N_DEV
N_STEP = N_DEV - 1


def kernel(x, router_W, route_idx, expert_W):
    def body(x_ref, rw_ref, idx_ref, ew_ref, out_ref, comm_ref,
             rs_send, rs_recv, ag_send, ag_recv):
        my = lax.axis_index("i")
        right = lax.rem(my + 1, N_DEV)

        xv = x_ref[:, :]
        scores = jnp.dot(xv, rw_ref[:, :], preferred_element_type=jnp.float32)
        smax = jnp.max(scores, axis=1, keepdims=True)
        ex = jnp.exp(scores - smax)
        probs = ex / jnp.sum(ex, axis=1, keepdims=True)
        idx0 = idx_ref[:, 0:1]
        idx1 = idx_ref[:, 1:2]
        eids = lax.broadcasted_iota(jnp.int32, (N_TOK, N_EXP), 1)
        p0 = jnp.sum(jnp.where(eids == idx0, probs, 0.0), axis=1, keepdims=True)
        p1 = jnp.sum(jnp.where(eids == idx1, probs, 0.0), axis=1, keepdims=True)
        denom = p0 + p1
        g0 = p0 / denom
        g1 = p1 / denom

        acc = jnp.zeros((N_TOK, D_OUT), jnp.float32)
        for el in range(E_LOCAL):
            gid = my * E_LOCAL + el
            ge = jnp.where(idx0 == gid, g0, 0.0) + jnp.where(idx1 == gid, g1, 0.0)
            acc = acc + jnp.dot(xv * ge, ew_ref[el],
                                preferred_element_type=jnp.float32)
        out_ref[:, :] = acc

        def cs(c):
            return pl.ds(c * CHUNK, CHUNK)

        for s in range(N_STEP):
            send_c = lax.rem(my - s + N_DEV, N_DEV)
            recv_c = lax.rem(my - s - 1 + N_DEV, N_DEV)
            if s == 0:
                src = out_ref.at[cs(send_c), :]
            else:
                src = comm_ref.at[s - 1]
            rdma = pltpu.make_async_remote_copy(
                src_ref=src,
                dst_ref=comm_ref.at[s],
                send_sem=rs_send.at[s],
                recv_sem=rs_recv.at[s],
                device_id=(right,),
                device_id_type=pl.DeviceIdType.MESH,
            )
            rdma.start()
            rdma.wait()
            comm_ref[s] = comm_ref[s] + out_ref[cs(recv_c), :]

        red_c = lax.rem(my + 1, N_DEV)
        out_ref[cs(red_c), :] = comm_ref[N_STEP - 1]

        for t in range(N_STEP):
            send_c = lax.rem(my + 1 - t + N_DEV, N_DEV)
            rdma = pltpu.make_async_remote_copy(
                src_ref=out_ref.at[cs(send_c), :],
                dst_ref=out_ref.at[cs(send_c), :],
                send_sem=ag_send.at[t],
                recv_sem=ag_recv.at[t],
                device_id=(right,),
                device_id_type=pl.DeviceIdType.MESH,
            )
            rdma.start()
            rdma.wait()

    return pl.pallas_call(
        body,
        out_shape=jax.ShapeDtypeStruct((N_TOK, D_OUT), jnp.float32),
        in_specs=[pl.BlockSpec(memory_space=pltpu.VMEM)] * 4,
        out_specs=pl.BlockSpec(memory_space=pltpu.VMEM),
        scratch_shapes=[
            pltpu.VMEM((N_STEP, CHUNK, D_OUT), jnp.float32),
            pltpu.SemaphoreType.DMA((N_STEP,)),
            pltpu.SemaphoreType.DMA((N_STEP,)),
            pltpu.SemaphoreType.DMA((N_STEP,)),
            pltpu.SemaphoreType.DMA((N_STEP,)),
        ],
        compiler_params=pltpu.CompilerParams(
            vmem_limit_bytes=100 * 1024 * 1024,
        ),
    )(x, router_W, route_idx, expert_W)


# device time: 154578 ns/iter; 1.8165x vs baseline; 1.8165x over previous
import jax
import jax.numpy as jnp
from jax import lax
from jax.experimental import pallas as pl
from jax.experimental.pallas import tpu as pltpu

N_DEV = 16
N_TOK = 2048
D_IN = 512
D_OUT = 1024
HALF = D_OUT // 2
E_LOCAL = 8
N_EXP = 128
P = 4
PCHUNK = N_TOK // P
P_STEP = P - 1


def kernel(x, router_W, route_idx, expert_W):
    def body(x_ref, rw_ref, idx_ref, ew_ref, out_ref,
             gates_ref, ownA, ownB, commA, commB,
             zworkA, zworkB, zr1A, zr1B, zr2A, zr2B,
             prsA_s, prsA_r, prsB_s, prsB_r,
             zrsA_s, zrsA_r, zrsB_s, zrsB_r,
             zagA_s, zagA_r, zagB_s, zagB_r,
             pagA_s, pagA_r, pagB_s, pagB_r):
        my = lax.axis_index("i")
        z = my // P
        j = lax.rem(my, P)
        p_right = z * P + lax.rem(j + 1, P)
        p_left = z * P + lax.rem(j - 1 + P, P)
        zp1 = jnp.bitwise_xor(z, 1) * P + j
        zp2 = jnp.bitwise_xor(z, 2) * P + j

        b0 = lax.rem(z, 2)
        b1 = z // 2
        keep1 = b0 * (PCHUNK // 2)
        send1 = (1 - b0) * (PCHUNK // 2)
        keep2 = keep1 + b1 * (PCHUNK // 4)
        send2 = keep1 + (1 - b1) * (PCHUNK // 4)

        scores = jnp.dot(x_ref[:, :], rw_ref[:, :],
                         preferred_element_type=jnp.float32)
        smax = jnp.max(scores, axis=1, keepdims=True)
        ex = jnp.exp(scores - smax)
        probs = ex / jnp.sum(ex, axis=1, keepdims=True)
        idx0 = idx_ref[:, 0:1]
        idx1 = idx_ref[:, 1:2]
        eids = lax.broadcasted_iota(jnp.int32, (N_TOK, N_EXP), 1)
        p0 = jnp.sum(jnp.where(eids == idx0, probs, 0.0), axis=1, keepdims=True)
        p1 = jnp.sum(jnp.where(eids == idx1, probs, 0.0), axis=1, keepdims=True)
        denom = p0 + p1
        g0 = p0 / denom
        g1 = p1 / denom
        cols = []
        for el in range(E_LOCAL):
            gid = my * E_LOCAL + el
            cols.append(jnp.where(idx0 == gid, g0, 0.0)
                        + jnp.where(idx1 == gid, g1, 0.0))
        gates_ref[:, :] = jnp.concatenate(cols, axis=1)

        def pcs(c):
            return pl.ds(c * PCHUNK, PCHUNK)

        def half_partial(c, col0):
            xs = x_ref[pcs(c), :]
            acc = jnp.zeros((PCHUNK, HALF), jnp.float32)
            for el in range(E_LOCAL):
                ge = gates_ref[pcs(c), el:el + 1]
                acc = acc + jnp.dot(xs * ge, ew_ref[el, :, col0:col0 + HALF],
                                    preferred_element_type=jnp.float32)
            return acc

        ownA[:, :] = half_partial(j, 0)
        barrier = pltpu.get_barrier_semaphore()
        for nbr in (p_left, p_right, zp1, zp2):
            pl.semaphore_signal(barrier, inc=1, device_id=(nbr,),
                                device_id_type=pl.DeviceIdType.MESH)
        pl.semaphore_wait(barrier, 4)

        def prs_rdma(s, srcA, srcB):
            rA = pltpu.make_async_remote_copy(
                src_ref=srcA, dst_ref=commA.at[s],
                send_sem=prsA_s.at[s], recv_sem=prsA_r.at[s],
                device_id=(p_right,), device_id_type=pl.DeviceIdType.MESH,
            )
            rB = pltpu.make_async_remote_copy(
                src_ref=srcB, dst_ref=commB.at[s],
                send_sem=prsB_s.at[s], recv_sem=prsB_r.at[s],
                device_id=(p_left,), device_id_type=pl.DeviceIdType.MESH,
            )
            return rA, rB

        rA, rB = prs_rdma(0, ownA, ownB)
        rA.start()
        ownB[:, :] = half_partial(j, HALF)
        rB.start()
        for s in range(P_STEP):
            pA = half_partial(lax.rem(j - s - 1 + P, P), 0)
            pB = half_partial(lax.rem(j + s + 1, P), HALF)
            rA.wait()
            commA[s] = commA[s] + pA
            rB.wait()
            commB[s] = commB[s] + pB
            if s + 1 < P_STEP:
                rA, rB = prs_rdma(s + 1, commA.at[s], commB.at[s])
                rA.start()
                rB.start()

        cA = lax.rem(j + 1, P)
        cB = lax.rem(j - 1 + P, P)
        zworkA[:, :] = commA[P_STEP - 1]
        zworkB[:, :] = commB[P_STEP - 1]

        def z_ex(w, r, ss, rs, stage, partner, off, n):
            return pltpu.make_async_remote_copy(
                src_ref=w.at[pl.ds(off, n), :], dst_ref=r,
                send_sem=ss.at[stage], recv_sem=rs.at[stage],
                device_id=(partner,), device_id_type=pl.DeviceIdType.MESH,
            )

        exA1 = z_ex(zworkA, zr1A, zrsA_s, zrsA_r, 0, zp1, send1, PCHUNK // 2)
        exB1 = z_ex(zworkB, zr1B, zrsB_s, zrsB_r, 0, zp1, send1, PCHUNK // 2)
        exA1.start()
        exB1.start()
        exA1.wait()
        zworkA[pl.ds(keep1, PCHUNK // 2), :] = (
            zworkA[pl.ds(keep1, PCHUNK // 2), :] + zr1A[:, :])
        exA2 = z_ex(zworkA, zr2A, zrsA_s, zrsA_r, 1, zp2, send2, PCHUNK // 4)
        exA2.start()
        exB1.wait()
        zworkB[pl.ds(keep1, PCHUNK // 2), :] = (
            zworkB[pl.ds(keep1, PCHUNK // 2), :] + zr1B[:, :])
        exB2 = z_ex(zworkB, zr2B, zrsB_s, zrsB_r, 1, zp2, send2, PCHUNK // 4)
        exB2.start()

        def zag(c, cl, ss, rs, stage, partner, off, n):
            sl = out_ref.at[pl.ds(c * PCHUNK + off, n), cl]
            return pltpu.make_async_remote_copy(
                src_ref=sl, dst_ref=sl,
                send_sem=ss.at[stage], recv_sem=rs.at[stage],
                device_id=(partner,), device_id_type=pl.DeviceIdType.MESH,
            )

        clA = slice(0, HALF)
        clB = slice(HALF, D_OUT)
        exA2.wait()
        out_ref[pl.ds(cA * PCHUNK + keep2, PCHUNK // 4), clA] = (
            zworkA[pl.ds(keep2, PCHUNK // 4), :] + zr2A[:, :])
        agA1 = zag(cA, clA, zagA_s, zagA_r, 0, zp2, keep2, PCHUNK // 4)
        agA1.start()
        agA2a = zag(cA, clA, zagA_s, zagA_r, 1, zp1, keep2, PCHUNK // 4)
        agA2a.start()
        exB2.wait()
        out_ref[pl.ds(cB * PCHUNK + keep2, PCHUNK // 4), clB] = (
            zworkB[pl.ds(keep2, PCHUNK // 4), :] + zr2B[:, :])
        agB1 = zag(cB, clB, zagB_s, zagB_r, 0, zp2, keep2, PCHUNK // 4)
        agB1.start()
        agB2a = zag(cB, clB, zagB_s, zagB_r, 1, zp1, keep2, PCHUNK // 4)
        agB2a.start()
        agA1.wait()
        agA2b = zag(cA, clA, zagA_s, zagA_r, 2, zp1, send2, PCHUNK // 4)
        agA2b.start()
        agB1.wait()
        agB2b = zag(cB, clB, zagB_s, zagB_r, 2, zp1, send2, PCHUNK // 4)
        agB2b.start()
        agA2a.wait()
        agA2b.wait()
        agB2a.wait()
        agB2b.wait()

        def pag(c, h, cl, ss, rs, v, partner):
            sl = out_ref.at[pl.ds(c * PCHUNK + h * (PCHUNK // 2), PCHUNK // 2), cl]
            return pltpu.make_async_remote_copy(
                src_ref=sl, dst_ref=sl,
                send_sem=ss.at[v], recv_sem=rs.at[v],
                device_id=(partner,), device_id_type=pl.DeviceIdType.MESH,
            )

        pend = {}
        for t in range(P_STEP):
            cA_t = lax.rem(j + 1 - t + P, P)
            cB_t = lax.rem(j - 1 + t + P, P)
            for h in range(2):
                if t >= 1:
                    wA, wB = pend.pop((t - 1, h))
                    wA.wait()
                    wB.wait()
                rA = pag(cA_t, h, clA, pagA_s, pagA_r, t * 2 + h, p_right)
                rB = pag(cB_t, h, clB, pagB_s, pagB_r, t * 2 + h, p_left)
                rA.start()
                rB.start()
                pend[(t, h)] = (rA, rB)
        for key in sorted(pend):
            wA, wB = pend[key]
            wA.wait()
            wB.wait()

    return pl.pallas_call(
        body,
        out_shape=jax.ShapeDtypeStruct((N_TOK, D_OUT), jnp.float32),
        in_specs=[pl.BlockSpec(memory_space=pltpu.VMEM)] * 4,
        out_specs=pl.BlockSpec(memory_space=pltpu.VMEM),
        scratch_shapes=[
            pltpu.VMEM((N_TOK, E_LOCAL), jnp.float32),
            pltpu.VMEM((PCHUNK, HALF), jnp.float32),
            pltpu.VMEM((PCHUNK, HALF), jnp.float32),
            pltpu.VMEM((P_STEP, PCHUNK, HALF), jnp.float32),
            pltpu.VMEM((P_STEP, PCHUNK, HALF), jnp.float32),
            pltpu.VMEM((PCHUNK, HALF), jnp.float32),
            pltpu.VMEM((PCHUNK, HALF), jnp.float32),
            pltpu.VMEM((PCHUNK // 2, HALF), jnp.float32),
            pltpu.VMEM((PCHUNK // 2, HALF), jnp.float32),
            pltpu.VMEM((PCHUNK // 4, HALF), jnp.float32),
            pltpu.VMEM((PCHUNK // 4, HALF), jnp.float32),
            pltpu.SemaphoreType.DMA((P_STEP,)),
            pltpu.SemaphoreType.DMA((P_STEP,)),
            pltpu.SemaphoreType.DMA((P_STEP,)),
            pltpu.SemaphoreType.DMA((P_STEP,)),
            pltpu.SemaphoreType.DMA((2,)),
            pltpu.SemaphoreType.DMA((2,)),
            pltpu.SemaphoreType.DMA((2,)),
            pltpu.SemaphoreType.DMA((2,)),
            pltpu.SemaphoreType.DMA((3,)),
            pltpu.SemaphoreType.DMA((3,)),
            pltpu.SemaphoreType.DMA((3,)),
            pltpu.SemaphoreType.DMA((3,)),
            pltpu.SemaphoreType.DMA((2 * P_STEP,)),
            pltpu.SemaphoreType.DMA((2 * P_STEP,)),
            pltpu.SemaphoreType.DMA((2 * P_STEP,)),
            pltpu.SemaphoreType.DMA((2 * P_STEP,)),
        ],
        compiler_params=pltpu.CompilerParams(
            vmem_limit_bytes=100 * 1024 * 1024,
            collective_id=0,
        ),
    )(x, router_W, route_idx, expert_W)
